# Initial kernel scaffold; baseline (speedup 1.0000x reference)
import jax
import jax.numpy as jnp
from jax.experimental import pallas as pl


def kernel(x, adj_t, label_p, cm, W_lin0, b_lin0, W_lin1, b_lin1, conv_weights):
    raise NotImplementedError("write your pallas kernel here")


# speedup vs baseline: 4.9944x; 4.9944x over previous
import functools
from math import log

import jax
import jax.numpy as jnp
from jax import lax
from jax.experimental import pallas as pl
from jax.experimental.pallas import tpu as pltpu
from jax.experimental.pallas import tpu_sc as plsc

N_NODES = 10000
N_EDGES = 320000
D = 128
NUM_LAYERS = 4
ALPHA = 0.1
THETA = 0.5

NC = 2
NS = 16
NW = NC * NS
CHUNK = 128
EPT = N_EDGES // NW
NCHUNK = -(-EPT // CHUNK)
EPT_PAD = NCHUNK * CHUNK
E_PAD = NW * EPT_PAD
SP_ROWS = 10240
RPT = SP_ROWS // NS
DUMMY_ROW = N_NODES

_mesh = plsc.VectorSubcoreMesh(
    core_axis_name="c", subcore_axis_name="s", num_cores=NC, num_subcores=NS)


@functools.partial(
    pl.kernel,
    out_type=jax.ShapeDtypeStruct((NC, SP_ROWS, D), jnp.float32),
    mesh=_mesh,
    scratch_types=[
        pltpu.VMEM((NCHUNK, CHUNK), jnp.int32),
        pltpu.VMEM((NCHUNK, CHUNK), jnp.int32),
        pltpu.VMEM((CHUNK, D), jnp.float32),
        pltpu.VMEM_SHARED((SP_ROWS, D), jnp.float32),
        pltpu.SemaphoreType.DMA,
    ],
)
def _sc_agg(h_hbm, src_hbm, dst_hbm, zeros_hbm, out_hbm,
            srcv, dstv, buf, agg, sem):
    cid = lax.axis_index("c")
    sid = lax.axis_index("s")
    t = cid * NS + sid
    pltpu.sync_copy(src_hbm.at[t], srcv)
    pltpu.sync_copy(dst_hbm.at[t], dstv)
    pltpu.sync_copy(zeros_hbm.at[pl.ds(sid * RPT, RPT)],
                    agg.at[pl.ds(sid * RPT, RPT)])
    plsc.subcore_barrier()

    def body(j, carry):
        pltpu.async_copy(h_hbm.at[srcv.at[j]], buf, sem).wait()
        pltpu.sync_copy(buf, agg.at[dstv.at[j]], add=True)
        return carry

    lax.fori_loop(0, NCHUNK, body, 0)
    plsc.subcore_barrier()
    pltpu.sync_copy(agg.at[pl.ds(sid * RPT, RPT)],
                    out_hbm.at[cid, pl.ds(sid * RPT, RPT)])


_BR = 2000


def _dense_in(x, W, b):
    def body(x_ref, w_ref, b_ref, o_ref):
        acc = jnp.dot(x_ref[...], w_ref[...],
                      preferred_element_type=jnp.float32)
        o_ref[...] = jnp.maximum(acc + b_ref[...], 0.0)
    return pl.pallas_call(
        body,
        grid=(N_NODES // _BR,),
        in_specs=[pl.BlockSpec((_BR, D), lambda i: (i, 0)),
                  pl.BlockSpec((D, D), lambda i: (0, 0)),
                  pl.BlockSpec((1, D), lambda i: (0, 0))],
        out_specs=pl.BlockSpec((_BR, D), lambda i: (i, 0)),
        out_shape=jax.ShapeDtypeStruct((N_NODES, D), jnp.float32),
    )(x, W, b.reshape(1, D))


def _dense_layer(p, x0, W, beta):
    def body(p_ref, x0_ref, w_ref, o_ref):
        t = (1.0 - ALPHA) * (p_ref[0] + p_ref[1]) + ALPHA * x0_ref[...]
        u = (1.0 - beta) * t + beta * jnp.dot(
            t, w_ref[...], preferred_element_type=jnp.float32)
        o_ref[...] = jnp.maximum(u, 0.0)
    return pl.pallas_call(
        body,
        grid=(N_NODES // _BR,),
        in_specs=[pl.BlockSpec((NC, _BR, D), lambda i: (0, i, 0)),
                  pl.BlockSpec((_BR, D), lambda i: (i, 0)),
                  pl.BlockSpec((D, D), lambda i: (0, 0))],
        out_specs=pl.BlockSpec((_BR, D), lambda i: (i, 0)),
        out_shape=jax.ShapeDtypeStruct((N_NODES, D), jnp.float32),
    )(p, x0, W)


def _dense_out(h, W, b):
    def body(h_ref, w_ref, b_ref, o_ref):
        acc = jnp.dot(h_ref[...], w_ref[...],
                      preferred_element_type=jnp.float32)
        o_ref[...] = acc + b_ref[...]
    return pl.pallas_call(
        body,
        grid=(N_NODES // _BR,),
        in_specs=[pl.BlockSpec((_BR, D), lambda i: (i, 0)),
                  pl.BlockSpec((D, D), lambda i: (0, 0)),
                  pl.BlockSpec((1, D), lambda i: (0, 0))],
        out_specs=pl.BlockSpec((_BR, D), lambda i: (i, 0)),
        out_shape=jax.ShapeDtypeStruct((N_NODES, D), jnp.float32),
    )(h, W, b.reshape(1, D))


def kernel(x, adj_t, label_p, cm, W_lin0, b_lin0, W_lin1, b_lin1,
           conv_weights):
    del label_p, cm
    src = adj_t[0].astype(jnp.int32)
    dst = adj_t[1].astype(jnp.int32)
    pad = E_PAD - N_EDGES
    srcp = jnp.concatenate(
        [src, jnp.zeros((pad,), jnp.int32)]).reshape(NW, NCHUNK, CHUNK)
    dstp = jnp.concatenate(
        [dst, jnp.full((pad,), DUMMY_ROW, jnp.int32)]).reshape(
            NW, NCHUNK, CHUNK)
    zeros = jnp.zeros((SP_ROWS, D), jnp.float32)

    h = _dense_in(x, W_lin0, b_lin0)
    x0 = h
    for i in range(NUM_LAYERS):
        beta = log(THETA / (i + 1) + 1.0)
        p = _sc_agg(h, srcp, dstp, zeros)
        h = _dense_layer(p, x0, conv_weights[i], beta)
    return _dense_out(h, W_lin1, b_lin1)
